# baseline (speedup 1.0000x reference)
import jax
import jax.numpy as jnp
from jax import lax
from jax.experimental import pallas as pl
from jax.experimental.pallas import tpu as pltpu
from jax.experimental.pallas import tpu_sc as plsc

M = 8192
T = 8
K = 2048
NC = 2
NS = 16
NW = NC * NS
TOK_PER_W = M // NW
TOK_CHUNK = 2
ROWS_PER_CHUNK = TOK_CHUNK * T
CHUNKS = TOK_PER_W // TOK_CHUNK
NBUF = 3
OUT_BLK = 2
HI = -65536


def _worker_id():
    return lax.axis_index("s") * NC + lax.axis_index("c")


def _body(expert_hbm, w_hbm, idx_hbm, out_hbm, idx_v, idxh_v, w_v,
          buf0, buf1, buf2, out_buf, sem0, sem1, sem2, semo):
    wid = _worker_id()
    tok_base = wid * TOK_PER_W
    w_pairs = expert_hbm.bitcast(jnp.int32)
    out_pairs = out_hbm.bitcast(jnp.int32)
    qbase = wid * (TOK_PER_W // 2)
    bufs = (buf0, buf1, buf2)
    sems = (sem0, sem1, sem2)

    pltpu.sync_copy(idx_hbm.at[pl.ds(tok_base * T, TOK_PER_W * T)], idx_v)
    pltpu.sync_copy(w_hbm.at[pl.ds(tok_base * T, TOK_PER_W * T)], w_v)

    def half_body(i, _):
        v = idx_v[pl.ds(i * 16, 16)]
        idxh_v[pl.ds(i * 16, 16)] = lax.shift_right_logical(v, 1)
        return 0

    lax.fori_loop(0, TOK_PER_W * T // 16, half_body, 0)

    def gather(g, b):
        idx = idxh_v.at[pl.ds(g * ROWS_PER_CHUNK, ROWS_PER_CHUNK)]
        return pltpu.make_async_copy(w_pairs.at[idx], bufs[b], sems[b])

    def out_blk_copy(blk, half):
        return pltpu.make_async_copy(
            out_buf.at[half],
            out_pairs.at[pl.ds(qbase + blk * OUT_BLK, OUT_BLK)], semo)

    def compute(g, buf, half):
        iv = idx_v[pl.ds(g * ROWS_PER_CHUNK, ROWS_PER_CHUNK)]
        wv = w_v[pl.ds(g * ROWS_PER_CHUNK, ROWS_PER_CHUNK)]
        wvec = []
        for j in range(ROWS_PER_CHUNK):
            pf = lax.convert_element_type(iv[j] & 1, jnp.float32)
            w_hi = wv[j] * pf
            w_lo = wv[j] - w_hi
            wvec.append(plsc.pack(lax.broadcast(w_lo, (16,)),
                                  lax.broadcast(w_hi, (16,)),
                                  format=plsc.PackFormat.INTERLEAVED))
        rb = g % OUT_BLK

        def ld(t, cc):
            return plsc.bitcast(buf[t, pl.ds(cc * 16, 16)], jnp.bfloat16)

        def col_half(cc):
            acc0 = ld(0, cc) * wvec[0]
            acc1 = ld(T, cc) * wvec[T]
            for t in range(1, T):
                acc0 = acc0 + ld(t, cc) * wvec[t]
                acc1 = acc1 + ld(T + t, cc) * wvec[T + t]
            a0, b0 = plsc.unpack(acc0, format=plsc.PackFormat.INTERLEAVED)
            a1, b1 = plsc.unpack(acc1, format=plsc.PackFormat.INTERLEAVED)
            s0 = a0 + b0
            s1 = a1 + b1
            r0 = plsc.bitcast(s0, jnp.int32)
            r0 = r0 + 0x7FFF + (lax.shift_right_logical(r0, 16) & 1)
            r1 = plsc.bitcast(s1, jnp.int32)
            r1 = r1 + 0x7FFF + (lax.shift_right_logical(r1, 16) & 1)
            word = lax.shift_right_logical(r0, 16) | (r1 & HI)
            out_buf[half, rb, pl.ds(cc * 16, 16)] = word

        def col_body(c2, _):
            col_half(2 * c2)
            col_half(2 * c2 + 1)
            return 0

        lax.fori_loop(0, K // 32, col_body, 0)

    gather(0, 0).start()
    gather(1, 1).start()

    def chunk_body(g, _):
        for b in range(NBUF):
            @pl.when(g % NBUF == b)
            def _(b=b):
                gather(g, b).wait()

                @pl.when(g + 2 < CHUNKS)
                def _():
                    gather(g + 2, (b + 2) % NBUF).start()

                blk = g // OUT_BLK
                half = blk % 2

                @pl.when((g % OUT_BLK == 0) & (blk >= 2))
                def _():
                    out_blk_copy(blk - 2, half).wait()

                compute(g, bufs[b], half)

                @pl.when(g % OUT_BLK == OUT_BLK - 1)
                def _():
                    out_blk_copy(blk, half).start()

        return 0

    lax.fori_loop(0, CHUNKS, chunk_body, 0)
    n_blk = CHUNKS // OUT_BLK
    out_blk_copy(n_blk - 2, (n_blk - 2) % 2).wait()
    out_blk_copy(n_blk - 1, (n_blk - 1) % 2).wait()


@jax.jit
def kernel(expert_output, topk_vals, inv_perm):
    w_f32 = topk_vals.astype(jnp.float32).reshape(M * T)

    mesh = plsc.VectorSubcoreMesh(core_axis_name="c", subcore_axis_name="s",
                                  num_cores=NC, num_subcores=NS)
    run = pl.kernel(
        _body,
        out_type=jax.ShapeDtypeStruct((M, K), jnp.bfloat16),
        mesh=mesh,
        compiler_params=pltpu.CompilerParams(needs_layout_passes=False),
        scratch_types=[
            pltpu.VMEM((TOK_PER_W * T,), jnp.int32),
            pltpu.VMEM((TOK_PER_W * T,), jnp.int32),
            pltpu.VMEM((TOK_PER_W * T,), jnp.float32),
            pltpu.VMEM((ROWS_PER_CHUNK, K), jnp.int32),
            pltpu.VMEM((ROWS_PER_CHUNK, K), jnp.int32),
            pltpu.VMEM((ROWS_PER_CHUNK, K), jnp.int32),
            pltpu.VMEM((2, OUT_BLK, K), jnp.int32),
            pltpu.SemaphoreType.DMA,
            pltpu.SemaphoreType.DMA,
            pltpu.SemaphoreType.DMA,
            pltpu.SemaphoreType.DMA,
        ],
    )
    return run(expert_output, w_f32, inv_perm)


# speedup vs baseline: 1.1776x; 1.1776x over previous
import jax
import jax.numpy as jnp
from jax import lax
from jax.experimental import pallas as pl
from jax.experimental.pallas import tpu as pltpu
from jax.experimental.pallas import tpu_sc as plsc

M = 8192
T = 8
K = 2048
NC = 2
NS = 16
NW = NC * NS
TOK_PER_W = M // NW
TOK_CHUNK = 2
ROWS_PER_CHUNK = TOK_CHUNK * T
CHUNKS = TOK_PER_W // TOK_CHUNK
NBUF = 3
OUT_BLK = 2
HI = -65536


def _worker_id():
    return lax.axis_index("s") * NC + lax.axis_index("c")


def _body(expert_hbm, w_hbm, idx_hbm, out_hbm, idx_v, idxh_v, w_v,
          buf0, buf1, buf2, out_buf, sem0, sem1, sem2, semo):
    wid = _worker_id()
    tok_base = wid * TOK_PER_W
    w_pairs = expert_hbm.bitcast(jnp.int32)
    out_pairs = out_hbm.bitcast(jnp.int32)
    qbase = wid * (TOK_PER_W // 2)
    bufs = (buf0, buf1, buf2)
    sems = (sem0, sem1, sem2)

    pltpu.sync_copy(idx_hbm.at[pl.ds(tok_base * T, TOK_PER_W * T)], idx_v)
    pltpu.sync_copy(w_hbm.at[pl.ds(tok_base * T, TOK_PER_W * T)], w_v)

    def half_body(i, _):
        v = idx_v[pl.ds(i * 16, 16)]
        idxh_v[pl.ds(i * 16, 16)] = lax.shift_right_logical(v, 1)
        return 0

    lax.fori_loop(0, TOK_PER_W * T // 16, half_body, 0)

    def gather(g, b):
        idx = idxh_v.at[pl.ds(g * ROWS_PER_CHUNK, ROWS_PER_CHUNK)]
        return pltpu.make_async_copy(w_pairs.at[idx], bufs[b], sems[b])

    def out_blk_copy(blk, half):
        return pltpu.make_async_copy(
            out_buf.at[half],
            out_pairs.at[pl.ds(qbase + blk * OUT_BLK, OUT_BLK)], semo)

    def compute(g, buf, half):
        iv = idx_v[pl.ds(g * ROWS_PER_CHUNK, ROWS_PER_CHUNK)]
        wv = w_v[pl.ds(g * ROWS_PER_CHUNK, ROWS_PER_CHUNK)]
        wvec = []
        for j in range(ROWS_PER_CHUNK):
            pf = lax.convert_element_type(iv[j] & 1, jnp.float32)
            w_hi = wv[j] * pf
            w_lo = wv[j] - w_hi
            wvec.append(plsc.pack(lax.broadcast(w_lo, (16,)),
                                  lax.broadcast(w_hi, (16,)),
                                  format=plsc.PackFormat.INTERLEAVED))
        rb = g % OUT_BLK

        def ld(t, cc):
            return plsc.bitcast(buf[t, pl.ds(cc * 16, 16)], jnp.bfloat16)

        def col_half(cc):
            acc0 = ld(0, cc) * wvec[0]
            acc1 = ld(T, cc) * wvec[T]
            for t in range(1, T):
                acc0 = acc0 + ld(t, cc) * wvec[t]
                acc1 = acc1 + ld(T + t, cc) * wvec[T + t]
            a0, b0 = plsc.unpack(acc0, format=plsc.PackFormat.INTERLEAVED)
            a1, b1 = plsc.unpack(acc1, format=plsc.PackFormat.INTERLEAVED)
            s0 = a0 + b0
            s1 = a1 + b1
            word = plsc.bitcast(
                plsc.pack(s0, s1, format=plsc.PackFormat.INTERLEAVED),
                jnp.int32)
            out_buf[half, rb, pl.ds(cc * 16, 16)] = word

        def col_body(cc, _):
            col_half(cc)
            return 0

        lax.fori_loop(0, K // 16, col_body, 0)

    gather(0, 0).start()
    gather(1, 1).start()

    def chunk_body(g, _):
        for b in range(NBUF):
            @pl.when(g % NBUF == b)
            def _(b=b):
                gather(g, b).wait()

                @pl.when(g + 2 < CHUNKS)
                def _():
                    gather(g + 2, (b + 2) % NBUF).start()

                blk = g // OUT_BLK
                half = blk % 2

                @pl.when((g % OUT_BLK == 0) & (blk >= 2))
                def _():
                    out_blk_copy(blk - 2, half).wait()

                compute(g, bufs[b], half)

                @pl.when(g % OUT_BLK == OUT_BLK - 1)
                def _():
                    out_blk_copy(blk, half).start()

        return 0

    lax.fori_loop(0, CHUNKS, chunk_body, 0)
    n_blk = CHUNKS // OUT_BLK
    out_blk_copy(n_blk - 2, (n_blk - 2) % 2).wait()
    out_blk_copy(n_blk - 1, (n_blk - 1) % 2).wait()


@jax.jit
def kernel(expert_output, topk_vals, inv_perm):
    w_f32 = topk_vals.astype(jnp.float32).reshape(M * T)

    mesh = plsc.VectorSubcoreMesh(core_axis_name="c", subcore_axis_name="s",
                                  num_cores=NC, num_subcores=NS)
    run = pl.kernel(
        _body,
        out_type=jax.ShapeDtypeStruct((M, K), jnp.bfloat16),
        mesh=mesh,
        compiler_params=pltpu.CompilerParams(needs_layout_passes=False),
        scratch_types=[
            pltpu.VMEM((TOK_PER_W * T,), jnp.int32),
            pltpu.VMEM((TOK_PER_W * T,), jnp.int32),
            pltpu.VMEM((TOK_PER_W * T,), jnp.float32),
            pltpu.VMEM((ROWS_PER_CHUNK, K), jnp.int32),
            pltpu.VMEM((ROWS_PER_CHUNK, K), jnp.int32),
            pltpu.VMEM((ROWS_PER_CHUNK, K), jnp.int32),
            pltpu.VMEM((2, OUT_BLK, K), jnp.int32),
            pltpu.SemaphoreType.DMA,
            pltpu.SemaphoreType.DMA,
            pltpu.SemaphoreType.DMA,
            pltpu.SemaphoreType.DMA,
        ],
    )
    return run(expert_output, w_f32, inv_perm)


# speedup vs baseline: 1.7220x; 1.4623x over previous
import jax
import jax.numpy as jnp
from jax import lax
from jax.experimental import pallas as pl
from jax.experimental.pallas import tpu as pltpu
from jax.experimental.pallas import tpu_sc as plsc

M = 8192
T = 8
K = 2048
NC = 2
NS = 16
NW = NC * NS
TOK_PER_W = M // NW
TOK_CHUNK = 2
ROWS_PER_CHUNK = TOK_CHUNK * T
CHUNKS = TOK_PER_W // TOK_CHUNK
NBUF = 3
OUT_BLK = 2
HI = -65536


def _worker_id():
    return lax.axis_index("s") * NC + lax.axis_index("c")


def _body(expert_hbm, w_hbm, idx_hbm, out_hbm, idx_v, idxh_v, w_v,
          buf0, buf1, buf2, out_buf, sem0, sem1, sem2, semo):
    wid = _worker_id()
    tok_base = wid * TOK_PER_W
    w_pairs = expert_hbm.bitcast(jnp.int32)
    out_pairs = out_hbm.bitcast(jnp.int32)
    qbase = wid * (TOK_PER_W // 2)
    bufs = (buf0, buf1, buf2)
    sems = (sem0, sem1, sem2)

    pltpu.sync_copy(idx_hbm.at[pl.ds(tok_base * T, TOK_PER_W * T)], idx_v)
    pltpu.sync_copy(w_hbm.at[pl.ds(tok_base * T, TOK_PER_W * T)], w_v)

    def half_body(i, _):
        v = idx_v[pl.ds(i * 16, 16)]
        idxh_v[pl.ds(i * 16, 16)] = lax.shift_right_logical(v, 1)
        return 0

    lax.fori_loop(0, TOK_PER_W * T // 16, half_body, 0)

    def gather(g, b):
        idx = idxh_v.at[pl.ds(g * ROWS_PER_CHUNK, ROWS_PER_CHUNK)]
        return pltpu.make_async_copy(w_pairs.at[idx], bufs[b], sems[b])

    def out_blk_copy(blk, half):
        return pltpu.make_async_copy(
            out_buf.at[half],
            out_pairs.at[pl.ds(qbase + blk * OUT_BLK, OUT_BLK)], semo)

    def compute(g, buf, half):
        iv = idx_v[pl.ds(g * ROWS_PER_CHUNK, ROWS_PER_CHUNK)]
        wv = w_v[pl.ds(g * ROWS_PER_CHUNK, ROWS_PER_CHUNK)]
        wvec = []
        for j in range(ROWS_PER_CHUNK):
            pf = lax.convert_element_type(iv[j] & 1, jnp.float32)
            w_hi = wv[j] * pf
            w_lo = wv[j] - w_hi
            wvec.append(plsc.pack(lax.broadcast(w_lo, (16,)),
                                  lax.broadcast(w_hi, (16,)),
                                  format=plsc.PackFormat.INTERLEAVED))
        rb = g % OUT_BLK

        def ld(t, cc):
            return plsc.bitcast(buf[t, pl.ds(cc * 16, 16)], jnp.bfloat16)

        def col_half(cc):
            acc0 = ld(0, cc) * wvec[0]
            acc1 = ld(T, cc) * wvec[T]
            for t in range(1, T):
                acc0 = acc0 + ld(t, cc) * wvec[t]
                acc1 = acc1 + ld(T + t, cc) * wvec[T + t]
            a0, b0 = plsc.unpack(acc0, format=plsc.PackFormat.INTERLEAVED)
            a1, b1 = plsc.unpack(acc1, format=plsc.PackFormat.INTERLEAVED)
            s0 = a0 + b0
            s1 = a1 + b1
            word = plsc.bitcast(
                plsc.pack(s0, s1, format=plsc.PackFormat.INTERLEAVED),
                jnp.int32)
            out_buf[half, rb, pl.ds(cc * 16, 16)] = word

        @plsc.parallel_loop(0, K // 16, unroll=2)
        def _(cc):
            col_half(cc)

    gather(0, 0).start()
    gather(1, 1).start()

    def chunk_body(g, _):
        for b in range(NBUF):
            @pl.when(g % NBUF == b)
            def _(b=b):
                gather(g, b).wait()

                @pl.when(g + 2 < CHUNKS)
                def _():
                    gather(g + 2, (b + 2) % NBUF).start()

                blk = g // OUT_BLK
                half = blk % 2

                @pl.when((g % OUT_BLK == 0) & (blk >= 2))
                def _():
                    out_blk_copy(blk - 2, half).wait()

                compute(g, bufs[b], half)

                @pl.when(g % OUT_BLK == OUT_BLK - 1)
                def _():
                    out_blk_copy(blk, half).start()

        return 0

    lax.fori_loop(0, CHUNKS, chunk_body, 0)
    n_blk = CHUNKS // OUT_BLK
    out_blk_copy(n_blk - 2, (n_blk - 2) % 2).wait()
    out_blk_copy(n_blk - 1, (n_blk - 1) % 2).wait()


@jax.jit
def kernel(expert_output, topk_vals, inv_perm):
    w_f32 = topk_vals.astype(jnp.float32).reshape(M * T)

    mesh = plsc.VectorSubcoreMesh(core_axis_name="c", subcore_axis_name="s",
                                  num_cores=NC, num_subcores=NS)
    run = pl.kernel(
        _body,
        out_type=jax.ShapeDtypeStruct((M, K), jnp.bfloat16),
        mesh=mesh,
        compiler_params=pltpu.CompilerParams(needs_layout_passes=False),
        scratch_types=[
            pltpu.VMEM((TOK_PER_W * T,), jnp.int32),
            pltpu.VMEM((TOK_PER_W * T,), jnp.int32),
            pltpu.VMEM((TOK_PER_W * T,), jnp.float32),
            pltpu.VMEM((ROWS_PER_CHUNK, K), jnp.int32),
            pltpu.VMEM((ROWS_PER_CHUNK, K), jnp.int32),
            pltpu.VMEM((ROWS_PER_CHUNK, K), jnp.int32),
            pltpu.VMEM((2, OUT_BLK, K), jnp.int32),
            pltpu.SemaphoreType.DMA,
            pltpu.SemaphoreType.DMA,
            pltpu.SemaphoreType.DMA,
            pltpu.SemaphoreType.DMA,
        ],
    )
    return run(expert_output, w_f32, inv_perm)
